# baseline (speedup 1.0000x reference)
import functools

import jax
import jax.numpy as jnp
from jax import lax
from jax.experimental import pallas as pl
from jax.experimental.pallas import tpu as pltpu
from jax.experimental.pallas import tpu_sc as plsc

N = 10000
S = 2500
D = 128
K = 32
RADIUS = 0.25

FPS_ROWS = 80
CHUNK = 1024


def _fps_body(start, end, px_ref, py_ref, pz_ref, d_in, vm_in, ar_in,
              idx_ref, ox_ref, oy_ref, oz_ref, d_out, vm_out, ar_out):
    px = px_ref[:]
    py = py_ref[:]
    pz = pz_ref[:]
    cshape = (8, 128)
    fshape = (FPS_ROWS, 128)
    flin = (lax.broadcasted_iota(jnp.int32, fshape, 0) * 128
            + lax.broadcasted_iota(jnp.int32, fshape, 1))
    clin = (lax.broadcasted_iota(jnp.int32, cshape, 0) * 128
            + lax.broadcasted_iota(jnp.int32, cshape, 1))
    czero_i = jnp.zeros(cshape, jnp.int32)
    czero_f = jnp.zeros(cshape, jnp.float32)
    sub8 = lax.broadcasted_iota(jnp.int32, cshape, 0)
    lane8 = lax.broadcasted_iota(jnp.int32, cshape, 1)
    lane1 = lax.broadcasted_iota(jnp.int32, (1, 128), 1)
    if start == 0:
        dists0 = jnp.where(flin < N, jnp.inf, -1.0).astype(jnp.float32)
        vmax0 = jnp.full(cshape, jnp.inf, jnp.float32)
        argrow0 = jnp.zeros(cshape, jnp.int32)
    else:
        dists0 = d_in[:]
        vmax0 = vm_in[:]
        argrow0 = ar_in[:]
    NEGINF = jnp.float32(-jnp.inf)
    BIG = jnp.int32(2**30)
    NV = FPS_ROWS // 8

    def body(i, carry):
        dists, vmax, argrow, cidx, cx, cy, cz = carry
        m = jnp.max(vmax)
        lincand = jnp.where(vmax == m,
                            (argrow * 8 + sub8) * 128 + lane8, BIG)
        nxt = jnp.min(lincand)
        row = lax.div(nxt, 128)
        onec = lane1 == lax.rem(nxt, 128)
        qx = jnp.sum(jnp.where(onec, px_ref[pl.ds(row, 1), :], 0.0))
        qy = jnp.sum(jnp.where(onec, py_ref[pl.ds(row, 1), :], 0.0))
        qz = jnp.sum(jnp.where(onec, pz_ref[pl.ds(row, 1), :], 0.0))
        nd = []
        vm = jnp.full(cshape, NEGINF)
        ar = czero_i
        for r in range(NV):
            sl = slice(r * 8, (r + 1) * 8)
            dx = px[sl] - qx
            dy = py[sl] - qy
            dz = pz[sl] - qz
            dn = (dx * dx + dz * dz) + dy * dy
            dr = jnp.minimum(dists[sl], dn)
            nd.append(dr)
            upd = dr > vm
            vm = jnp.where(upd, dr, vm)
            ar = jnp.where(upd, r, ar)
        dists = jnp.concatenate(nd, axis=0)
        slot = lax.rem(i - start, CHUNK)
        hit = clin == slot
        cidx = jnp.where(hit, nxt, cidx)
        cx = jnp.where(hit, qx, cx)
        cy = jnp.where(hit, qy, cy)
        cz = jnp.where(hit, qz, cz)

        @pl.when((slot == CHUNK - 1) | (i == end - 1))
        def _flush():
            base = lax.div(i - start, CHUNK) * 8
            idx_ref[pl.ds(base, 8), :] = cidx
            ox_ref[pl.ds(base, 8), :] = cx
            oy_ref[pl.ds(base, 8), :] = cy
            oz_ref[pl.ds(base, 8), :] = cz

        return dists, vm, ar, cidx, cx, cy, cz

    dists, vm, ar, _, _, _, _ = lax.fori_loop(
        start, end, body, (dists0, vmax0, argrow0,
                           czero_i, czero_f, czero_f, czero_f))
    d_out[:] = dists
    vm_out[:] = vm
    ar_out[:] = ar


WAVES = ((0, 1024), (1024, 2048), (2048, 2500))


def _run_fps_wave(coords, state, start, end, interpret=False):
    f32, i32 = jnp.float32, jnp.int32
    rows = 8 * ((end - start + CHUNK - 1) // CHUNK)
    out_shapes = (
        jax.ShapeDtypeStruct((rows, 128), i32),
        jax.ShapeDtypeStruct((rows, 128), f32),
        jax.ShapeDtypeStruct((rows, 128), f32),
        jax.ShapeDtypeStruct((rows, 128), f32),
        jax.ShapeDtypeStruct((FPS_ROWS, 128), f32),
        jax.ShapeDtypeStruct((8, 128), f32),
        jax.ShapeDtypeStruct((8, 128), i32),
    )
    res = pl.pallas_call(
        functools.partial(_fps_body, start, end),
        out_shape=out_shapes,
        interpret=interpret,
    )(*coords, *state)
    return res[:4], res[4:]


def _fps_coords(pos):
    pad = FPS_ROWS * 128 - N
    big = jnp.float32(1e9)
    coords = []
    for c in range(3):
        col = jnp.concatenate([pos[:, c], jnp.full((pad,), big, jnp.float32)])
        coords.append(col.reshape(FPS_ROWS, 128))
    return coords


def _fps_state0():
    shape = (FPS_ROWS, 128)
    lin = (lax.broadcasted_iota(jnp.int32, shape, 0) * 128
           + lax.broadcasted_iota(jnp.int32, shape, 1))
    dists0 = jnp.where(lin < N, jnp.inf, -1.0).astype(jnp.float32)
    vmax0 = jnp.full((8, 128), jnp.inf, jnp.float32)
    argrow0 = jnp.zeros((8, 128), jnp.int32)
    return dists0, vmax0, argrow0


def _run_fps(pos, interpret=False):
    coords = _fps_coords(pos)
    state = _fps_state0()
    idxs, xs, ys, zs = [], [], [], []
    for start, end in WAVES:
        (io, ox, oy, oz), state = _run_fps_wave(
            coords, state, start, end, interpret)
        idxs.append(io.reshape(-1)[:end - start])
        xs.append(ox.reshape(-1)[:end - start])
        ys.append(oy.reshape(-1)[:end - start])
        zs.append(oz.reshape(-1)[:end - start])
    fps_idx = jnp.concatenate(idxs)
    pos_sub = jnp.stack([jnp.concatenate(xs), jnp.concatenate(ys),
                         jnp.concatenate(zs)], axis=1)
    return fps_idx, pos_sub


SP = 2560
QB = 32
NW = 32
QW = SP // NW
LANES = 16
CAP = N // LANES
R2 = RADIUS * RADIUS


def _precomp_body(xp_ref, w_ref, g_ref):
    g_ref[:] = jnp.dot(xp_ref[:], w_ref[:],
                       preferred_element_type=jnp.float32,
                       precision=lax.Precision.HIGHEST)


def _run_precomp(x, pos, W1, interpret=False):
    xp = jnp.concatenate(
        [x, pos, jnp.zeros((N, 5), jnp.float32)], axis=1)
    w = jnp.concatenate([W1, jnp.zeros((5, 128), jnp.float32)], axis=0)
    return pl.pallas_call(
        _precomp_body,
        out_shape=jax.ShapeDtypeStruct((N, 128), jnp.float32),
        interpret=interpret,
    )(xp, w)


NP = 10112
BIGJ = N
UNROLL = 8


def _nbr_body(qw, px_hbm, py_hbm, pz_hbm, psx_hbm, psy_hbm, psz_hbm,
              nbr_hbm, pxv, pyv, pzv, qxv, qyv, qzv, cj, stage):
    wid = lax.axis_index("s") * 2 + lax.axis_index("c")
    pltpu.sync_copy(px_hbm, pxv)
    pltpu.sync_copy(py_hbm, pyv)
    pltpu.sync_copy(pz_hbm, pzv)
    pltpu.sync_copy(psx_hbm.at[pl.ds(wid * qw, qw)], qxv.at[pl.ds(0, qw)])
    pltpu.sync_copy(psy_hbm.at[pl.ds(wid * qw, qw)], qyv.at[pl.ds(0, qw)])
    pltpu.sync_copy(psz_hbm.at[pl.ds(wid * qw, qw)], qzv.at[pl.ds(0, qw)])
    lane = lax.iota(jnp.int32, LANES)
    lane0 = lane == 0
    INF = jnp.float32(jnp.inf)
    BIGI = jnp.int32(2**30)

    def qbody(q, _):
        qb = (q // LANES) * LANES
        qsel = lane == q - qb
        qx = jnp.full((LANES,),
                      jnp.sum(jnp.where(qsel, qxv[pl.ds(qb, LANES)], 0.0)))
        qy = jnp.full((LANES,),
                      jnp.sum(jnp.where(qsel, qyv[pl.ds(qb, LANES)], 0.0)))
        qz = jnp.full((LANES,),
                      jnp.sum(jnp.where(qsel, qzv[pl.ds(qb, LANES)], 0.0)))

        @plsc.parallel_loop(0, N, step=LANES, unroll=UNROLL,
                            carry=jnp.zeros((LANES,), jnp.int32))
        def lcnt(base, lc):
            dx = pxv[pl.ds(base, LANES)] - qx
            dy = pyv[pl.ds(base, LANES)] - qy
            dz = pzv[pl.ds(base, LANES)] - qz
            d2 = (dx * dx + dy * dy) + dz * dz
            msk = d2 <= R2
            addr = lc * LANES + lane
            plsc.store_scatter(cj, [addr], base + lane, mask=msk)
            return lc + msk.astype(jnp.int32)

        maxc = jnp.max(lcnt)

        def ext_body(k, fill):
            def row_body(cc, st):
                bd, bj, ba = st
                base = cc * LANES
                jr = cj[pl.ds(base, LANES)]
                js = jnp.minimum(jnp.maximum(jr, 0), jnp.int32(NP - 1))
                dxj = plsc.load_gather(pxv, [js]) - qx
                dyj = plsc.load_gather(pyv, [js]) - qy
                dzj = plsc.load_gather(pzv, [js]) - qz
                d2j = (dxj * dxj + dyj * dyj) + dzj * dzj
                d = jnp.where(cc < lcnt, d2j, INF)
                better = (d < bd) | ((d == bd) & (jr < bj))
                return (jnp.where(better, d, bd),
                        jnp.where(better, jr, bj),
                        jnp.where(better, base + lane, ba))

            bd, bj, ba = lax.fori_loop(
                0, maxc, row_body,
                (jnp.full((LANES,), INF),
                 jnp.full((LANES,), BIGI),
                 jnp.zeros((LANES,), jnp.int32)))
            m = jnp.min(bd)
            elig = bd == m
            jm = jnp.min(jnp.where(elig, bj, BIGI))
            am = jnp.min(jnp.where(elig & (bj == jm), ba, BIGI))
            found = m <= R2
            am_s = jnp.where(found, am, 0)
            plsc.store_scatter(cj, [jnp.full((LANES,), am_s, jnp.int32)],
                               jnp.full((LANES,), BIGJ), mask=lane0)
            fill = jnp.where((k == 0) & found, jm, fill)
            jout = jnp.where(found, jm, fill)
            plsc.store_scatter(stage,
                               [jnp.full((LANES,), q * K + k, jnp.int32)],
                               jnp.full((LANES,), jout, jnp.int32),
                               mask=lane0)
            return fill

        lax.fori_loop(0, K, ext_body, jnp.int32(0))
        return 0

    lax.fori_loop(0, qw, qbody, 0)
    pltpu.sync_copy(stage, nbr_hbm.at[pl.ds(wid * qw * K, qw * K)])


def _run_nbr(px, py, pz, psx, psy, psz, qw):
    mesh = plsc.VectorSubcoreMesh(core_axis_name="c", subcore_axis_name="s")
    f32, i32 = jnp.float32, jnp.int32
    kfn = functools.partial(
        pl.kernel, mesh=mesh,
        compiler_params=pltpu.CompilerParams(needs_layout_passes=False),
        out_type=jax.ShapeDtypeStruct((NW * qw * K,), i32),
        scratch_types=[
            pltpu.VMEM((NP,), f32), pltpu.VMEM((NP,), f32),
            pltpu.VMEM((NP,), f32),
            pltpu.VMEM((128,), f32), pltpu.VMEM((128,), f32),
            pltpu.VMEM((128,), f32),
            pltpu.VMEM((CAP * LANES + 128,), i32),
            pltpu.VMEM((qw * K,), i32),
        ],
    )(functools.partial(_nbr_body, qw))
    return kfn(px, py, pz, psx, psy, psz)


def _gather_body(qw, gch, do_batch,
                 g_hbm, nbr_hbm, fidx_hbm, batch_hbm, gg_hbm, bsub_hbm,
                 idxv, rows0, rows1, bvec, fvec, bout,
                 gs0, gs1, os0, os1):
    NCH = qw * K // gch
    GCH = gch
    wid = lax.axis_index("s") * 2 + lax.axis_index("c")
    base = wid * qw * K
    pltpu.sync_copy(nbr_hbm.at[pl.ds(base, qw * K)], idxv)

    bufs = (rows0, rows1)
    gsems = (gs0, gs1)
    osems = (os0, os1)
    gets = [None, None]
    outs = [None, None]
    gets[0] = pltpu.async_copy(
        g_hbm.at[idxv.at[pl.ds(0, GCH)]], bufs[0], gsems[0])
    for i in range(NCH):
        p = i % 2
        gets[p].wait()
        if i + 1 < NCH:
            q = (i + 1) % 2
            if outs[q] is not None:
                outs[q].wait()
            gets[q] = pltpu.async_copy(
                g_hbm.at[idxv.at[pl.ds((i + 1) * GCH, GCH)]],
                bufs[q], gsems[q])
        outs[p] = pltpu.async_copy(
            bufs[p], gg_hbm.at[pl.ds(base + i * GCH, GCH)], osems[p])
    for p in (0, 1):
        if outs[p] is not None:
            outs[p].wait()

    if do_batch:
        @pl.when(wid == 0)
        def _batch():
            pltpu.sync_copy(batch_hbm, bvec)
            pltpu.sync_copy(fidx_hbm, fvec)

            def bb(b, _):
                iv = fvec[pl.ds(b * LANES, LANES)]
                bout[pl.ds(b * LANES, LANES)] = plsc.load_gather(bvec, [iv])
                return 0

            lax.fori_loop(0, SP // LANES, bb, 0)
            pltpu.sync_copy(bout, bsub_hbm)


def _run_gather(g, nbr_flat, fidx_pad, batch, qw, do_batch):
    mesh = plsc.VectorSubcoreMesh(core_axis_name="c", subcore_axis_name="s")
    f32, i32 = jnp.float32, jnp.int32
    gch = min(qw * K // 4, 320)
    kfn = functools.partial(
        pl.kernel, mesh=mesh,
        compiler_params=pltpu.CompilerParams(needs_layout_passes=False),
        out_type=(jax.ShapeDtypeStruct((NW * qw * K, 128), f32),
                  jax.ShapeDtypeStruct((SP,), i32)),
        scratch_types=[
            pltpu.VMEM((qw * K,), i32),
            pltpu.VMEM((gch, 128), f32),
            pltpu.VMEM((gch, 128), f32),
            pltpu.VMEM((N,), i32),
            pltpu.VMEM((SP,), i32),
            pltpu.VMEM((SP,), i32),
            pltpu.SemaphoreType.DMA, pltpu.SemaphoreType.DMA,
            pltpu.SemaphoreType.DMA, pltpu.SemaphoreType.DMA,
        ],
    )(functools.partial(_gather_body, qw, gch, do_batch))
    return kfn(g, nbr_flat, fidx_pad, batch)


def _mlp_body(gg_ref, ps_ref, w1p_ref, b1_ref, w2_ref, b2_ref, out_ref):
    t = jnp.dot(ps_ref[:], w1p_ref[:],
                preferred_element_type=jnp.float32,
                precision=lax.Precision.HIGHEST)
    g3 = gg_ref[:].reshape(QB, K, 128)
    h1 = jnp.maximum(g3 - t[:, None, :] + b1_ref[:].reshape(1, 1, 128), 0.0)
    h2 = jnp.dot(h1.reshape(QB * K, 128), w2_ref[:],
                 preferred_element_type=jnp.float32,
                 precision=lax.Precision.HIGHEST) + b2_ref[:]
    out_ref[:] = jnp.max(h2.reshape(QB, K, 128), axis=1)


def _run_mlp(gg, ps_pad, W1, b1, W2, b2, sp_w, interpret=False):
    w1p = jnp.concatenate(
        [W1[D:D + 3], jnp.zeros((5, 128), jnp.float32)], axis=0)
    nblk = sp_w // QB
    return pl.pallas_call(
        _mlp_body,
        grid=(nblk,),
        in_specs=[
            pl.BlockSpec((QB * K, 128), lambda i: (i, 0)),
            pl.BlockSpec((QB, 8), lambda i: (i, 0)),
            pl.BlockSpec((8, 128), lambda i: (0, 0)),
            pl.BlockSpec((1, 128), lambda i: (0, 0)),
            pl.BlockSpec((128, 128), lambda i: (0, 0)),
            pl.BlockSpec((1, 128), lambda i: (0, 0)),
        ],
        out_specs=pl.BlockSpec((QB, 128), lambda i: (i, 0)),
        out_shape=jax.ShapeDtypeStruct((sp_w, 128), jnp.float32),
        interpret=interpret,
    )(gg, ps_pad, w1p, b1.reshape(1, 128), W2, b2.reshape(1, 128))


def kernel(x, pos, batch, W1, b1, W2, b2):
    g = _run_precomp(x, pos, W1)
    big = jnp.float32(1e9)
    padp = jnp.full((NP - N,), big, jnp.float32)
    pxp = jnp.concatenate([pos[:, 0], padp])
    pyp = jnp.concatenate([pos[:, 1], padp])
    pzp = jnp.concatenate([pos[:, 2], padp])

    coords = _fps_coords(pos)
    state = _fps_state0()
    idxs, xs, ys, zs, outs = [], [], [], [], []
    bsub = None
    waves_out = []
    for start, end in WAVES:
        (io, ox, oy, oz), state = _run_fps_wave(coords, state, start, end)
        wlen = end - start
        waves_out.append((io.reshape(-1)[:wlen], ox.reshape(-1)[:wlen],
                          oy.reshape(-1)[:wlen], oz.reshape(-1)[:wlen]))
    for iof, oxf, oyf, ozf in waves_out:
        idxs.append(iof)
        xs.append(oxf)
        ys.append(oyf)
        zs.append(ozf)
    nbrs = []
    for wi, (iof, oxf, oyf, ozf) in enumerate(waves_out):
        wlen = iof.shape[0]
        sp_w = 1024 if wlen > 512 else 512
        qw = sp_w // NW
        padq = jnp.full((sp_w - wlen,), big, jnp.float32)
        nbrs.append(_run_nbr(pxp, pyp, pzp,
                             jnp.concatenate([oxf, padq]),
                             jnp.concatenate([oyf, padq]),
                             jnp.concatenate([ozf, padq]), qw))
    for wi, (iof, oxf, oyf, ozf) in enumerate(waves_out):
        wlen = iof.shape[0]
        sp_w = 1024 if wlen > 512 else 512
        qw = sp_w // NW
        last = wi == len(WAVES) - 1
        if last:
            fidx_pad = jnp.pad(jnp.concatenate(idxs), (0, SP - S))
        else:
            fidx_pad = jnp.zeros((SP,), jnp.int32)
        gg_w, bs_w = _run_gather(g, nbrs[wi], fidx_pad, batch, qw, last)
        if last:
            bsub = bs_w
        ps_pad = jnp.pad(
            jnp.stack([oxf, oyf, ozf], axis=1),
            ((0, sp_w - wlen), (0, 5)))
        outs.append(_run_mlp(gg_w, ps_pad, W1, b1, W2, b2, sp_w)[:wlen])
    pos_sub = jnp.stack([jnp.concatenate(xs), jnp.concatenate(ys),
                         jnp.concatenate(zs)], axis=1)
    out = jnp.concatenate(outs)
    return (out, pos_sub, bsub[:S])


# speedup vs baseline: 1.0010x; 1.0010x over previous
import functools

import jax
import jax.numpy as jnp
from jax import lax
from jax.experimental import pallas as pl
from jax.experimental.pallas import tpu as pltpu
from jax.experimental.pallas import tpu_sc as plsc

N = 10000
S = 2500
D = 128
K = 32
RADIUS = 0.25

FPS_ROWS = 80
CHUNK = 1024


def _fps_body(start, end, px_ref, py_ref, pz_ref, d_in, vm_in, ar_in,
              idx_ref, ox_ref, oy_ref, oz_ref, d_out, vm_out, ar_out):
    px = px_ref[:]
    py = py_ref[:]
    pz = pz_ref[:]
    cshape = (8, 128)
    fshape = (FPS_ROWS, 128)
    flin = (lax.broadcasted_iota(jnp.int32, fshape, 0) * 128
            + lax.broadcasted_iota(jnp.int32, fshape, 1))
    clin = (lax.broadcasted_iota(jnp.int32, cshape, 0) * 128
            + lax.broadcasted_iota(jnp.int32, cshape, 1))
    czero_i = jnp.zeros(cshape, jnp.int32)
    czero_f = jnp.zeros(cshape, jnp.float32)
    sub8 = lax.broadcasted_iota(jnp.int32, cshape, 0)
    lane8 = lax.broadcasted_iota(jnp.int32, cshape, 1)
    lane1 = lax.broadcasted_iota(jnp.int32, (1, 128), 1)
    if start == 0:
        dists0 = jnp.where(flin < N, jnp.inf, -1.0).astype(jnp.float32)
        vmax0 = jnp.full(cshape, jnp.inf, jnp.float32)
        argrow0 = jnp.zeros(cshape, jnp.int32)
    else:
        dists0 = d_in[:]
        vmax0 = vm_in[:]
        argrow0 = ar_in[:]
    NEGINF = jnp.float32(-jnp.inf)
    BIG = jnp.int32(2**30)
    NV = FPS_ROWS // 8

    def body(i, carry):
        dists, vmax, argrow, cidx, cx, cy, cz = carry
        m = jnp.max(vmax)
        lincand = jnp.where(vmax == m,
                            (argrow * 8 + sub8) * 128 + lane8, BIG)
        nxt = jnp.min(lincand)
        row = lax.div(nxt, 128)
        onec = lane1 == lax.rem(nxt, 128)
        qx = jnp.sum(jnp.where(onec, px_ref[pl.ds(row, 1), :], 0.0))
        qy = jnp.sum(jnp.where(onec, py_ref[pl.ds(row, 1), :], 0.0))
        qz = jnp.sum(jnp.where(onec, pz_ref[pl.ds(row, 1), :], 0.0))
        nd = []
        vm = jnp.full(cshape, NEGINF)
        ar = czero_i
        for r in range(NV):
            sl = slice(r * 8, (r + 1) * 8)
            dx = px[sl] - qx
            dy = py[sl] - qy
            dz = pz[sl] - qz
            dn = (dx * dx + dz * dz) + dy * dy
            dr = jnp.minimum(dists[sl], dn)
            nd.append(dr)
            upd = dr > vm
            vm = jnp.where(upd, dr, vm)
            ar = jnp.where(upd, r, ar)
        dists = jnp.concatenate(nd, axis=0)
        slot = lax.rem(i - start, CHUNK)
        hit = clin == slot
        cidx = jnp.where(hit, nxt, cidx)
        cx = jnp.where(hit, qx, cx)
        cy = jnp.where(hit, qy, cy)
        cz = jnp.where(hit, qz, cz)

        @pl.when((slot == CHUNK - 1) | (i == end - 1))
        def _flush():
            base = lax.div(i - start, CHUNK) * 8
            idx_ref[pl.ds(base, 8), :] = cidx
            ox_ref[pl.ds(base, 8), :] = cx
            oy_ref[pl.ds(base, 8), :] = cy
            oz_ref[pl.ds(base, 8), :] = cz

        return dists, vm, ar, cidx, cx, cy, cz

    dists, vm, ar, _, _, _, _ = lax.fori_loop(
        start, end, body, (dists0, vmax0, argrow0,
                           czero_i, czero_f, czero_f, czero_f))
    d_out[:] = dists
    vm_out[:] = vm
    ar_out[:] = ar


WAVES = ((0, 1024), (1024, 2048), (2048, 2500))


def _run_fps_wave(coords, state, start, end, interpret=False):
    f32, i32 = jnp.float32, jnp.int32
    rows = 8 * ((end - start + CHUNK - 1) // CHUNK)
    out_shapes = (
        jax.ShapeDtypeStruct((rows, 128), i32),
        jax.ShapeDtypeStruct((rows, 128), f32),
        jax.ShapeDtypeStruct((rows, 128), f32),
        jax.ShapeDtypeStruct((rows, 128), f32),
        jax.ShapeDtypeStruct((FPS_ROWS, 128), f32),
        jax.ShapeDtypeStruct((8, 128), f32),
        jax.ShapeDtypeStruct((8, 128), i32),
    )
    res = pl.pallas_call(
        functools.partial(_fps_body, start, end),
        out_shape=out_shapes,
        interpret=interpret,
    )(*coords, *state)
    return res[:4], res[4:]


def _fps_coords(pos):
    pad = FPS_ROWS * 128 - N
    big = jnp.float32(1e9)
    coords = []
    for c in range(3):
        col = jnp.concatenate([pos[:, c], jnp.full((pad,), big, jnp.float32)])
        coords.append(col.reshape(FPS_ROWS, 128))
    return coords


def _fps_state0():
    shape = (FPS_ROWS, 128)
    lin = (lax.broadcasted_iota(jnp.int32, shape, 0) * 128
           + lax.broadcasted_iota(jnp.int32, shape, 1))
    dists0 = jnp.where(lin < N, jnp.inf, -1.0).astype(jnp.float32)
    vmax0 = jnp.full((8, 128), jnp.inf, jnp.float32)
    argrow0 = jnp.zeros((8, 128), jnp.int32)
    return dists0, vmax0, argrow0


def _run_fps(pos, interpret=False):
    coords = _fps_coords(pos)
    state = _fps_state0()
    idxs, xs, ys, zs = [], [], [], []
    for start, end in WAVES:
        (io, ox, oy, oz), state = _run_fps_wave(
            coords, state, start, end, interpret)
        idxs.append(io.reshape(-1)[:end - start])
        xs.append(ox.reshape(-1)[:end - start])
        ys.append(oy.reshape(-1)[:end - start])
        zs.append(oz.reshape(-1)[:end - start])
    fps_idx = jnp.concatenate(idxs)
    pos_sub = jnp.stack([jnp.concatenate(xs), jnp.concatenate(ys),
                         jnp.concatenate(zs)], axis=1)
    return fps_idx, pos_sub


SP = 2560
QB = 32
NW = 32
QW = SP // NW
LANES = 16
CAP = N // LANES
R2 = RADIUS * RADIUS


def _precomp_body(xp_ref, w_ref, g_ref):
    g_ref[:] = jnp.dot(xp_ref[:], w_ref[:],
                       preferred_element_type=jnp.float32,
                       precision=lax.Precision.HIGHEST)


def _run_precomp(x, pos, W1, interpret=False):
    xp = jnp.concatenate(
        [x, pos, jnp.zeros((N, 5), jnp.float32)], axis=1)
    w = jnp.concatenate([W1, jnp.zeros((5, 128), jnp.float32)], axis=0)
    return pl.pallas_call(
        _precomp_body,
        out_shape=jax.ShapeDtypeStruct((N, 128), jnp.float32),
        interpret=interpret,
    )(xp, w)


NP = 10112
BIGJ = N
UNROLL = 16


def _nbr_body(qw, px_hbm, py_hbm, pz_hbm, psx_hbm, psy_hbm, psz_hbm,
              nbr_hbm, pxv, pyv, pzv, qxv, qyv, qzv, cj, stage):
    wid = lax.axis_index("s") * 2 + lax.axis_index("c")
    pltpu.sync_copy(px_hbm, pxv)
    pltpu.sync_copy(py_hbm, pyv)
    pltpu.sync_copy(pz_hbm, pzv)
    pltpu.sync_copy(psx_hbm.at[pl.ds(wid * qw, qw)], qxv.at[pl.ds(0, qw)])
    pltpu.sync_copy(psy_hbm.at[pl.ds(wid * qw, qw)], qyv.at[pl.ds(0, qw)])
    pltpu.sync_copy(psz_hbm.at[pl.ds(wid * qw, qw)], qzv.at[pl.ds(0, qw)])
    lane = lax.iota(jnp.int32, LANES)
    lane0 = lane == 0
    INF = jnp.float32(jnp.inf)
    BIGI = jnp.int32(2**30)

    def qbody(q, _):
        qb = (q // LANES) * LANES
        qsel = lane == q - qb
        qx = jnp.full((LANES,),
                      jnp.sum(jnp.where(qsel, qxv[pl.ds(qb, LANES)], 0.0)))
        qy = jnp.full((LANES,),
                      jnp.sum(jnp.where(qsel, qyv[pl.ds(qb, LANES)], 0.0)))
        qz = jnp.full((LANES,),
                      jnp.sum(jnp.where(qsel, qzv[pl.ds(qb, LANES)], 0.0)))

        @plsc.parallel_loop(0, N, step=LANES, unroll=UNROLL,
                            carry=jnp.zeros((LANES,), jnp.int32))
        def lcnt(base, lc):
            dx = pxv[pl.ds(base, LANES)] - qx
            dy = pyv[pl.ds(base, LANES)] - qy
            dz = pzv[pl.ds(base, LANES)] - qz
            d2 = (dx * dx + dy * dy) + dz * dz
            msk = d2 <= R2
            addr = lc * LANES + lane
            plsc.store_scatter(cj, [addr], base + lane, mask=msk)
            return lc + msk.astype(jnp.int32)

        maxc = jnp.max(lcnt)

        def ext_body(k, fill):
            def row_body(cc, st):
                bd, bj, ba = st
                base = cc * LANES
                jr = cj[pl.ds(base, LANES)]
                js = jnp.minimum(jnp.maximum(jr, 0), jnp.int32(NP - 1))
                dxj = plsc.load_gather(pxv, [js]) - qx
                dyj = plsc.load_gather(pyv, [js]) - qy
                dzj = plsc.load_gather(pzv, [js]) - qz
                d2j = (dxj * dxj + dyj * dyj) + dzj * dzj
                d = jnp.where(cc < lcnt, d2j, INF)
                better = (d < bd) | ((d == bd) & (jr < bj))
                return (jnp.where(better, d, bd),
                        jnp.where(better, jr, bj),
                        jnp.where(better, base + lane, ba))

            bd, bj, ba = lax.fori_loop(
                0, maxc, row_body,
                (jnp.full((LANES,), INF),
                 jnp.full((LANES,), BIGI),
                 jnp.zeros((LANES,), jnp.int32)))
            m = jnp.min(bd)
            elig = bd == m
            jm = jnp.min(jnp.where(elig, bj, BIGI))
            am = jnp.min(jnp.where(elig & (bj == jm), ba, BIGI))
            found = m <= R2
            am_s = jnp.where(found, am, 0)
            plsc.store_scatter(cj, [jnp.full((LANES,), am_s, jnp.int32)],
                               jnp.full((LANES,), BIGJ), mask=lane0)
            fill = jnp.where((k == 0) & found, jm, fill)
            jout = jnp.where(found, jm, fill)
            plsc.store_scatter(stage,
                               [jnp.full((LANES,), q * K + k, jnp.int32)],
                               jnp.full((LANES,), jout, jnp.int32),
                               mask=lane0)
            return fill

        lax.fori_loop(0, K, ext_body, jnp.int32(0))
        return 0

    lax.fori_loop(0, qw, qbody, 0)
    pltpu.sync_copy(stage, nbr_hbm.at[pl.ds(wid * qw * K, qw * K)])


def _run_nbr(px, py, pz, psx, psy, psz, qw):
    mesh = plsc.VectorSubcoreMesh(core_axis_name="c", subcore_axis_name="s")
    f32, i32 = jnp.float32, jnp.int32
    kfn = functools.partial(
        pl.kernel, mesh=mesh,
        compiler_params=pltpu.CompilerParams(needs_layout_passes=False),
        out_type=jax.ShapeDtypeStruct((NW * qw * K,), i32),
        scratch_types=[
            pltpu.VMEM((NP,), f32), pltpu.VMEM((NP,), f32),
            pltpu.VMEM((NP,), f32),
            pltpu.VMEM((128,), f32), pltpu.VMEM((128,), f32),
            pltpu.VMEM((128,), f32),
            pltpu.VMEM((CAP * LANES + 128,), i32),
            pltpu.VMEM((qw * K,), i32),
        ],
    )(functools.partial(_nbr_body, qw))
    return kfn(px, py, pz, psx, psy, psz)


def _gather_body(qw, gch, do_batch,
                 g_hbm, nbr_hbm, fidx_hbm, batch_hbm, gg_hbm, bsub_hbm,
                 idxv, rows0, rows1, bvec, fvec, bout,
                 gs0, gs1, os0, os1):
    NCH = qw * K // gch
    GCH = gch
    wid = lax.axis_index("s") * 2 + lax.axis_index("c")
    base = wid * qw * K
    pltpu.sync_copy(nbr_hbm.at[pl.ds(base, qw * K)], idxv)

    bufs = (rows0, rows1)
    gsems = (gs0, gs1)
    osems = (os0, os1)
    gets = [None, None]
    outs = [None, None]
    gets[0] = pltpu.async_copy(
        g_hbm.at[idxv.at[pl.ds(0, GCH)]], bufs[0], gsems[0])
    for i in range(NCH):
        p = i % 2
        gets[p].wait()
        if i + 1 < NCH:
            q = (i + 1) % 2
            if outs[q] is not None:
                outs[q].wait()
            gets[q] = pltpu.async_copy(
                g_hbm.at[idxv.at[pl.ds((i + 1) * GCH, GCH)]],
                bufs[q], gsems[q])
        outs[p] = pltpu.async_copy(
            bufs[p], gg_hbm.at[pl.ds(base + i * GCH, GCH)], osems[p])
    for p in (0, 1):
        if outs[p] is not None:
            outs[p].wait()

    if do_batch:
        @pl.when(wid == 0)
        def _batch():
            pltpu.sync_copy(batch_hbm, bvec)
            pltpu.sync_copy(fidx_hbm, fvec)

            def bb(b, _):
                iv = fvec[pl.ds(b * LANES, LANES)]
                bout[pl.ds(b * LANES, LANES)] = plsc.load_gather(bvec, [iv])
                return 0

            lax.fori_loop(0, SP // LANES, bb, 0)
            pltpu.sync_copy(bout, bsub_hbm)


def _run_gather(g, nbr_flat, fidx_pad, batch, qw, do_batch):
    mesh = plsc.VectorSubcoreMesh(core_axis_name="c", subcore_axis_name="s")
    f32, i32 = jnp.float32, jnp.int32
    gch = min(qw * K // 4, 320)
    kfn = functools.partial(
        pl.kernel, mesh=mesh,
        compiler_params=pltpu.CompilerParams(needs_layout_passes=False),
        out_type=(jax.ShapeDtypeStruct((NW * qw * K, 128), f32),
                  jax.ShapeDtypeStruct((SP,), i32)),
        scratch_types=[
            pltpu.VMEM((qw * K,), i32),
            pltpu.VMEM((gch, 128), f32),
            pltpu.VMEM((gch, 128), f32),
            pltpu.VMEM((N,), i32),
            pltpu.VMEM((SP,), i32),
            pltpu.VMEM((SP,), i32),
            pltpu.SemaphoreType.DMA, pltpu.SemaphoreType.DMA,
            pltpu.SemaphoreType.DMA, pltpu.SemaphoreType.DMA,
        ],
    )(functools.partial(_gather_body, qw, gch, do_batch))
    return kfn(g, nbr_flat, fidx_pad, batch)


def _mlp_body(gg_ref, ps_ref, w1p_ref, b1_ref, w2_ref, b2_ref, out_ref):
    t = jnp.dot(ps_ref[:], w1p_ref[:],
                preferred_element_type=jnp.float32,
                precision=lax.Precision.HIGHEST)
    g3 = gg_ref[:].reshape(QB, K, 128)
    h1 = jnp.maximum(g3 - t[:, None, :] + b1_ref[:].reshape(1, 1, 128), 0.0)
    h2 = jnp.dot(h1.reshape(QB * K, 128), w2_ref[:],
                 preferred_element_type=jnp.float32,
                 precision=lax.Precision.HIGHEST) + b2_ref[:]
    out_ref[:] = jnp.max(h2.reshape(QB, K, 128), axis=1)


def _run_mlp(gg, ps_pad, W1, b1, W2, b2, sp_w, interpret=False):
    w1p = jnp.concatenate(
        [W1[D:D + 3], jnp.zeros((5, 128), jnp.float32)], axis=0)
    nblk = sp_w // QB
    return pl.pallas_call(
        _mlp_body,
        grid=(nblk,),
        in_specs=[
            pl.BlockSpec((QB * K, 128), lambda i: (i, 0)),
            pl.BlockSpec((QB, 8), lambda i: (i, 0)),
            pl.BlockSpec((8, 128), lambda i: (0, 0)),
            pl.BlockSpec((1, 128), lambda i: (0, 0)),
            pl.BlockSpec((128, 128), lambda i: (0, 0)),
            pl.BlockSpec((1, 128), lambda i: (0, 0)),
        ],
        out_specs=pl.BlockSpec((QB, 128), lambda i: (i, 0)),
        out_shape=jax.ShapeDtypeStruct((sp_w, 128), jnp.float32),
        interpret=interpret,
    )(gg, ps_pad, w1p, b1.reshape(1, 128), W2, b2.reshape(1, 128))


def kernel(x, pos, batch, W1, b1, W2, b2):
    g = _run_precomp(x, pos, W1)
    big = jnp.float32(1e9)
    padp = jnp.full((NP - N,), big, jnp.float32)
    pxp = jnp.concatenate([pos[:, 0], padp])
    pyp = jnp.concatenate([pos[:, 1], padp])
    pzp = jnp.concatenate([pos[:, 2], padp])

    coords = _fps_coords(pos)
    state = _fps_state0()
    idxs, xs, ys, zs, outs = [], [], [], [], []
    bsub = None
    waves_out = []
    for start, end in WAVES:
        (io, ox, oy, oz), state = _run_fps_wave(coords, state, start, end)
        wlen = end - start
        waves_out.append((io.reshape(-1)[:wlen], ox.reshape(-1)[:wlen],
                          oy.reshape(-1)[:wlen], oz.reshape(-1)[:wlen]))
    for iof, oxf, oyf, ozf in waves_out:
        idxs.append(iof)
        xs.append(oxf)
        ys.append(oyf)
        zs.append(ozf)
    nbrs = []
    for wi, (iof, oxf, oyf, ozf) in enumerate(waves_out):
        wlen = iof.shape[0]
        sp_w = 1024 if wlen > 512 else 512
        qw = sp_w // NW
        padq = jnp.full((sp_w - wlen,), big, jnp.float32)
        nbrs.append(_run_nbr(pxp, pyp, pzp,
                             jnp.concatenate([oxf, padq]),
                             jnp.concatenate([oyf, padq]),
                             jnp.concatenate([ozf, padq]), qw))
    for wi, (iof, oxf, oyf, ozf) in enumerate(waves_out):
        wlen = iof.shape[0]
        sp_w = 1024 if wlen > 512 else 512
        qw = sp_w // NW
        last = wi == len(WAVES) - 1
        if last:
            fidx_pad = jnp.pad(jnp.concatenate(idxs), (0, SP - S))
        else:
            fidx_pad = jnp.zeros((SP,), jnp.int32)
        gg_w, bs_w = _run_gather(g, nbrs[wi], fidx_pad, batch, qw, last)
        if last:
            bsub = bs_w
        ps_pad = jnp.pad(
            jnp.stack([oxf, oyf, ozf], axis=1),
            ((0, sp_w - wlen), (0, 5)))
        outs.append(_run_mlp(gg_w, ps_pad, W1, b1, W2, b2, sp_w)[:wlen])
    pos_sub = jnp.stack([jnp.concatenate(xs), jnp.concatenate(ys),
                         jnp.concatenate(zs)], axis=1)
    out = jnp.concatenate(outs)
    return (out, pos_sub, bsub[:S])


# speedup vs baseline: 1.0011x; 1.0001x over previous
import functools

import jax
import jax.numpy as jnp
from jax import lax
from jax.experimental import pallas as pl
from jax.experimental.pallas import tpu as pltpu
from jax.experimental.pallas import tpu_sc as plsc

N = 10000
S = 2500
D = 128
K = 32
RADIUS = 0.25

FPS_ROWS = 80
CHUNK = 1024


def _fps_body(start, end, px_ref, py_ref, pz_ref, d_in, vm_in, ar_in,
              idx_ref, ox_ref, oy_ref, oz_ref, d_out, vm_out, ar_out):
    px = px_ref[:]
    py = py_ref[:]
    pz = pz_ref[:]
    cshape = (8, 128)
    fshape = (FPS_ROWS, 128)
    flin = (lax.broadcasted_iota(jnp.int32, fshape, 0) * 128
            + lax.broadcasted_iota(jnp.int32, fshape, 1))
    clin = (lax.broadcasted_iota(jnp.int32, cshape, 0) * 128
            + lax.broadcasted_iota(jnp.int32, cshape, 1))
    czero_i = jnp.zeros(cshape, jnp.int32)
    czero_f = jnp.zeros(cshape, jnp.float32)
    sub8 = lax.broadcasted_iota(jnp.int32, cshape, 0)
    lane8 = lax.broadcasted_iota(jnp.int32, cshape, 1)
    lane1 = lax.broadcasted_iota(jnp.int32, (1, 128), 1)
    if start == 0:
        dists0 = jnp.where(flin < N, jnp.inf, -1.0).astype(jnp.float32)
        vmax0 = jnp.full(cshape, jnp.inf, jnp.float32)
        argrow0 = jnp.zeros(cshape, jnp.int32)
    else:
        dists0 = d_in[:]
        vmax0 = vm_in[:]
        argrow0 = ar_in[:]
    NEGINF = jnp.float32(-jnp.inf)
    BIG = jnp.int32(2**30)
    NV = FPS_ROWS // 8

    def body(i, carry):
        dists, vmax, argrow, cidx, cx, cy, cz = carry
        m = jnp.max(vmax, axis=(0, 1), keepdims=True)
        lincand = jnp.where(vmax == m,
                            (argrow * 8 + sub8) * 128 + lane8, BIG)
        nxt = jnp.min(lincand)
        row = lax.div(nxt, 128)
        onec = lane1 == lax.rem(nxt, 128)
        qx = jnp.sum(jnp.where(onec, px_ref[pl.ds(row, 1), :], 0.0),
                     axis=(0, 1), keepdims=True)
        qy = jnp.sum(jnp.where(onec, py_ref[pl.ds(row, 1), :], 0.0),
                     axis=(0, 1), keepdims=True)
        qz = jnp.sum(jnp.where(onec, pz_ref[pl.ds(row, 1), :], 0.0),
                     axis=(0, 1), keepdims=True)
        nd = []
        vm = jnp.full(cshape, NEGINF)
        ar = czero_i
        for r in range(NV):
            sl = slice(r * 8, (r + 1) * 8)
            dx = px[sl] - qx
            dy = py[sl] - qy
            dz = pz[sl] - qz
            dn = (dx * dx + dz * dz) + dy * dy
            dr = jnp.minimum(dists[sl], dn)
            nd.append(dr)
            upd = dr > vm
            vm = jnp.where(upd, dr, vm)
            ar = jnp.where(upd, r, ar)
        dists = jnp.concatenate(nd, axis=0)
        slot = lax.rem(i - start, CHUNK)
        hit = clin == slot
        cidx = jnp.where(hit, nxt, cidx)
        cx = jnp.where(hit, qx, cx)
        cy = jnp.where(hit, qy, cy)
        cz = jnp.where(hit, qz, cz)

        @pl.when((slot == CHUNK - 1) | (i == end - 1))
        def _flush():
            base = lax.div(i - start, CHUNK) * 8
            idx_ref[pl.ds(base, 8), :] = cidx
            ox_ref[pl.ds(base, 8), :] = cx
            oy_ref[pl.ds(base, 8), :] = cy
            oz_ref[pl.ds(base, 8), :] = cz

        return dists, vm, ar, cidx, cx, cy, cz

    dists, vm, ar, _, _, _, _ = lax.fori_loop(
        start, end, body, (dists0, vmax0, argrow0,
                           czero_i, czero_f, czero_f, czero_f))
    d_out[:] = dists
    vm_out[:] = vm
    ar_out[:] = ar


WAVES = ((0, 1024), (1024, 2048), (2048, 2500))


def _run_fps_wave(coords, state, start, end, interpret=False):
    f32, i32 = jnp.float32, jnp.int32
    rows = 8 * ((end - start + CHUNK - 1) // CHUNK)
    out_shapes = (
        jax.ShapeDtypeStruct((rows, 128), i32),
        jax.ShapeDtypeStruct((rows, 128), f32),
        jax.ShapeDtypeStruct((rows, 128), f32),
        jax.ShapeDtypeStruct((rows, 128), f32),
        jax.ShapeDtypeStruct((FPS_ROWS, 128), f32),
        jax.ShapeDtypeStruct((8, 128), f32),
        jax.ShapeDtypeStruct((8, 128), i32),
    )
    res = pl.pallas_call(
        functools.partial(_fps_body, start, end),
        out_shape=out_shapes,
        interpret=interpret,
    )(*coords, *state)
    return res[:4], res[4:]


def _fps_coords(pos):
    pad = FPS_ROWS * 128 - N
    big = jnp.float32(1e9)
    coords = []
    for c in range(3):
        col = jnp.concatenate([pos[:, c], jnp.full((pad,), big, jnp.float32)])
        coords.append(col.reshape(FPS_ROWS, 128))
    return coords


def _fps_state0():
    shape = (FPS_ROWS, 128)
    lin = (lax.broadcasted_iota(jnp.int32, shape, 0) * 128
           + lax.broadcasted_iota(jnp.int32, shape, 1))
    dists0 = jnp.where(lin < N, jnp.inf, -1.0).astype(jnp.float32)
    vmax0 = jnp.full((8, 128), jnp.inf, jnp.float32)
    argrow0 = jnp.zeros((8, 128), jnp.int32)
    return dists0, vmax0, argrow0


def _run_fps(pos, interpret=False):
    coords = _fps_coords(pos)
    state = _fps_state0()
    idxs, xs, ys, zs = [], [], [], []
    for start, end in WAVES:
        (io, ox, oy, oz), state = _run_fps_wave(
            coords, state, start, end, interpret)
        idxs.append(io.reshape(-1)[:end - start])
        xs.append(ox.reshape(-1)[:end - start])
        ys.append(oy.reshape(-1)[:end - start])
        zs.append(oz.reshape(-1)[:end - start])
    fps_idx = jnp.concatenate(idxs)
    pos_sub = jnp.stack([jnp.concatenate(xs), jnp.concatenate(ys),
                         jnp.concatenate(zs)], axis=1)
    return fps_idx, pos_sub


SP = 2560
QB = 32
NW = 32
QW = SP // NW
LANES = 16
CAP = N // LANES
R2 = RADIUS * RADIUS


def _precomp_body(xp_ref, w_ref, g_ref):
    g_ref[:] = jnp.dot(xp_ref[:], w_ref[:],
                       preferred_element_type=jnp.float32,
                       precision=lax.Precision.HIGHEST)


def _run_precomp(x, pos, W1, interpret=False):
    xp = jnp.concatenate(
        [x, pos, jnp.zeros((N, 5), jnp.float32)], axis=1)
    w = jnp.concatenate([W1, jnp.zeros((5, 128), jnp.float32)], axis=0)
    return pl.pallas_call(
        _precomp_body,
        out_shape=jax.ShapeDtypeStruct((N, 128), jnp.float32),
        interpret=interpret,
    )(xp, w)


NP = 10112
BIGJ = N
UNROLL = 16


def _nbr_body(qw, px_hbm, py_hbm, pz_hbm, psx_hbm, psy_hbm, psz_hbm,
              nbr_hbm, pxv, pyv, pzv, qxv, qyv, qzv, cj, stage):
    wid = lax.axis_index("s") * 2 + lax.axis_index("c")
    pltpu.sync_copy(px_hbm, pxv)
    pltpu.sync_copy(py_hbm, pyv)
    pltpu.sync_copy(pz_hbm, pzv)
    pltpu.sync_copy(psx_hbm.at[pl.ds(wid * qw, qw)], qxv.at[pl.ds(0, qw)])
    pltpu.sync_copy(psy_hbm.at[pl.ds(wid * qw, qw)], qyv.at[pl.ds(0, qw)])
    pltpu.sync_copy(psz_hbm.at[pl.ds(wid * qw, qw)], qzv.at[pl.ds(0, qw)])
    lane = lax.iota(jnp.int32, LANES)
    lane0 = lane == 0
    INF = jnp.float32(jnp.inf)
    BIGI = jnp.int32(2**30)

    def qbody(q, _):
        qb = (q // LANES) * LANES
        qsel = lane == q - qb
        qx = jnp.full((LANES,),
                      jnp.sum(jnp.where(qsel, qxv[pl.ds(qb, LANES)], 0.0)))
        qy = jnp.full((LANES,),
                      jnp.sum(jnp.where(qsel, qyv[pl.ds(qb, LANES)], 0.0)))
        qz = jnp.full((LANES,),
                      jnp.sum(jnp.where(qsel, qzv[pl.ds(qb, LANES)], 0.0)))

        @plsc.parallel_loop(0, N, step=LANES, unroll=UNROLL,
                            carry=jnp.zeros((LANES,), jnp.int32))
        def lcnt(base, lc):
            dx = pxv[pl.ds(base, LANES)] - qx
            dy = pyv[pl.ds(base, LANES)] - qy
            dz = pzv[pl.ds(base, LANES)] - qz
            d2 = (dx * dx + dy * dy) + dz * dz
            msk = d2 <= R2
            addr = lc * LANES + lane
            plsc.store_scatter(cj, [addr], base + lane, mask=msk)
            return lc + msk.astype(jnp.int32)

        maxc = jnp.max(lcnt)

        def ext_body(k, fill):
            def row_body(cc, st):
                bd, bj, ba = st
                base = cc * LANES
                jr = cj[pl.ds(base, LANES)]
                js = jnp.minimum(jnp.maximum(jr, 0), jnp.int32(NP - 1))
                dxj = plsc.load_gather(pxv, [js]) - qx
                dyj = plsc.load_gather(pyv, [js]) - qy
                dzj = plsc.load_gather(pzv, [js]) - qz
                d2j = (dxj * dxj + dyj * dyj) + dzj * dzj
                d = jnp.where(cc < lcnt, d2j, INF)
                better = (d < bd) | ((d == bd) & (jr < bj))
                return (jnp.where(better, d, bd),
                        jnp.where(better, jr, bj),
                        jnp.where(better, base + lane, ba))

            bd, bj, ba = lax.fori_loop(
                0, maxc, row_body,
                (jnp.full((LANES,), INF),
                 jnp.full((LANES,), BIGI),
                 jnp.zeros((LANES,), jnp.int32)))
            m = jnp.min(bd)
            elig = bd == m
            jm = jnp.min(jnp.where(elig, bj, BIGI))
            am = jnp.min(jnp.where(elig & (bj == jm), ba, BIGI))
            found = m <= R2
            am_s = jnp.where(found, am, 0)
            plsc.store_scatter(cj, [jnp.full((LANES,), am_s, jnp.int32)],
                               jnp.full((LANES,), BIGJ), mask=lane0)
            fill = jnp.where((k == 0) & found, jm, fill)
            jout = jnp.where(found, jm, fill)
            plsc.store_scatter(stage,
                               [jnp.full((LANES,), q * K + k, jnp.int32)],
                               jnp.full((LANES,), jout, jnp.int32),
                               mask=lane0)
            return fill

        lax.fori_loop(0, K, ext_body, jnp.int32(0))
        return 0

    lax.fori_loop(0, qw, qbody, 0)
    pltpu.sync_copy(stage, nbr_hbm.at[pl.ds(wid * qw * K, qw * K)])


def _run_nbr(px, py, pz, psx, psy, psz, qw):
    mesh = plsc.VectorSubcoreMesh(core_axis_name="c", subcore_axis_name="s")
    f32, i32 = jnp.float32, jnp.int32
    kfn = functools.partial(
        pl.kernel, mesh=mesh,
        compiler_params=pltpu.CompilerParams(needs_layout_passes=False),
        out_type=jax.ShapeDtypeStruct((NW * qw * K,), i32),
        scratch_types=[
            pltpu.VMEM((NP,), f32), pltpu.VMEM((NP,), f32),
            pltpu.VMEM((NP,), f32),
            pltpu.VMEM((128,), f32), pltpu.VMEM((128,), f32),
            pltpu.VMEM((128,), f32),
            pltpu.VMEM((CAP * LANES + 128,), i32),
            pltpu.VMEM((qw * K,), i32),
        ],
    )(functools.partial(_nbr_body, qw))
    return kfn(px, py, pz, psx, psy, psz)


def _gather_body(qw, gch, do_batch,
                 g_hbm, nbr_hbm, fidx_hbm, batch_hbm, gg_hbm, bsub_hbm,
                 idxv, rows0, rows1, bvec, fvec, bout,
                 gs0, gs1, os0, os1):
    NCH = qw * K // gch
    GCH = gch
    wid = lax.axis_index("s") * 2 + lax.axis_index("c")
    base = wid * qw * K
    pltpu.sync_copy(nbr_hbm.at[pl.ds(base, qw * K)], idxv)

    bufs = (rows0, rows1)
    gsems = (gs0, gs1)
    osems = (os0, os1)
    gets = [None, None]
    outs = [None, None]
    gets[0] = pltpu.async_copy(
        g_hbm.at[idxv.at[pl.ds(0, GCH)]], bufs[0], gsems[0])
    for i in range(NCH):
        p = i % 2
        gets[p].wait()
        if i + 1 < NCH:
            q = (i + 1) % 2
            if outs[q] is not None:
                outs[q].wait()
            gets[q] = pltpu.async_copy(
                g_hbm.at[idxv.at[pl.ds((i + 1) * GCH, GCH)]],
                bufs[q], gsems[q])
        outs[p] = pltpu.async_copy(
            bufs[p], gg_hbm.at[pl.ds(base + i * GCH, GCH)], osems[p])
    for p in (0, 1):
        if outs[p] is not None:
            outs[p].wait()

    if do_batch:
        @pl.when(wid == 0)
        def _batch():
            pltpu.sync_copy(batch_hbm, bvec)
            pltpu.sync_copy(fidx_hbm, fvec)

            def bb(b, _):
                iv = fvec[pl.ds(b * LANES, LANES)]
                bout[pl.ds(b * LANES, LANES)] = plsc.load_gather(bvec, [iv])
                return 0

            lax.fori_loop(0, SP // LANES, bb, 0)
            pltpu.sync_copy(bout, bsub_hbm)


def _run_gather(g, nbr_flat, fidx_pad, batch, qw, do_batch):
    mesh = plsc.VectorSubcoreMesh(core_axis_name="c", subcore_axis_name="s")
    f32, i32 = jnp.float32, jnp.int32
    gch = min(qw * K // 4, 320)
    kfn = functools.partial(
        pl.kernel, mesh=mesh,
        compiler_params=pltpu.CompilerParams(needs_layout_passes=False),
        out_type=(jax.ShapeDtypeStruct((NW * qw * K, 128), f32),
                  jax.ShapeDtypeStruct((SP,), i32)),
        scratch_types=[
            pltpu.VMEM((qw * K,), i32),
            pltpu.VMEM((gch, 128), f32),
            pltpu.VMEM((gch, 128), f32),
            pltpu.VMEM((N,), i32),
            pltpu.VMEM((SP,), i32),
            pltpu.VMEM((SP,), i32),
            pltpu.SemaphoreType.DMA, pltpu.SemaphoreType.DMA,
            pltpu.SemaphoreType.DMA, pltpu.SemaphoreType.DMA,
        ],
    )(functools.partial(_gather_body, qw, gch, do_batch))
    return kfn(g, nbr_flat, fidx_pad, batch)


def _mlp_body(gg_ref, ps_ref, w1p_ref, b1_ref, w2_ref, b2_ref, out_ref):
    t = jnp.dot(ps_ref[:], w1p_ref[:],
                preferred_element_type=jnp.float32,
                precision=lax.Precision.HIGHEST)
    g3 = gg_ref[:].reshape(QB, K, 128)
    h1 = jnp.maximum(g3 - t[:, None, :] + b1_ref[:].reshape(1, 1, 128), 0.0)
    h2 = jnp.dot(h1.reshape(QB * K, 128), w2_ref[:],
                 preferred_element_type=jnp.float32,
                 precision=lax.Precision.HIGHEST) + b2_ref[:]
    out_ref[:] = jnp.max(h2.reshape(QB, K, 128), axis=1)


def _run_mlp(gg, ps_pad, W1, b1, W2, b2, sp_w, interpret=False):
    w1p = jnp.concatenate(
        [W1[D:D + 3], jnp.zeros((5, 128), jnp.float32)], axis=0)
    nblk = sp_w // QB
    return pl.pallas_call(
        _mlp_body,
        grid=(nblk,),
        in_specs=[
            pl.BlockSpec((QB * K, 128), lambda i: (i, 0)),
            pl.BlockSpec((QB, 8), lambda i: (i, 0)),
            pl.BlockSpec((8, 128), lambda i: (0, 0)),
            pl.BlockSpec((1, 128), lambda i: (0, 0)),
            pl.BlockSpec((128, 128), lambda i: (0, 0)),
            pl.BlockSpec((1, 128), lambda i: (0, 0)),
        ],
        out_specs=pl.BlockSpec((QB, 128), lambda i: (i, 0)),
        out_shape=jax.ShapeDtypeStruct((sp_w, 128), jnp.float32),
        interpret=interpret,
    )(gg, ps_pad, w1p, b1.reshape(1, 128), W2, b2.reshape(1, 128))


def kernel(x, pos, batch, W1, b1, W2, b2):
    g = _run_precomp(x, pos, W1)
    big = jnp.float32(1e9)
    padp = jnp.full((NP - N,), big, jnp.float32)
    pxp = jnp.concatenate([pos[:, 0], padp])
    pyp = jnp.concatenate([pos[:, 1], padp])
    pzp = jnp.concatenate([pos[:, 2], padp])

    coords = _fps_coords(pos)
    state = _fps_state0()
    idxs, xs, ys, zs, outs = [], [], [], [], []
    bsub = None
    waves_out = []
    for start, end in WAVES:
        (io, ox, oy, oz), state = _run_fps_wave(coords, state, start, end)
        wlen = end - start
        waves_out.append((io.reshape(-1)[:wlen], ox.reshape(-1)[:wlen],
                          oy.reshape(-1)[:wlen], oz.reshape(-1)[:wlen]))
    for iof, oxf, oyf, ozf in waves_out:
        idxs.append(iof)
        xs.append(oxf)
        ys.append(oyf)
        zs.append(ozf)
    nbrs = []
    for wi, (iof, oxf, oyf, ozf) in enumerate(waves_out):
        wlen = iof.shape[0]
        sp_w = 1024 if wlen > 512 else 512
        qw = sp_w // NW
        padq = jnp.full((sp_w - wlen,), big, jnp.float32)
        nbrs.append(_run_nbr(pxp, pyp, pzp,
                             jnp.concatenate([oxf, padq]),
                             jnp.concatenate([oyf, padq]),
                             jnp.concatenate([ozf, padq]), qw))
    for wi, (iof, oxf, oyf, ozf) in enumerate(waves_out):
        wlen = iof.shape[0]
        sp_w = 1024 if wlen > 512 else 512
        qw = sp_w // NW
        last = wi == len(WAVES) - 1
        if last:
            fidx_pad = jnp.pad(jnp.concatenate(idxs), (0, SP - S))
        else:
            fidx_pad = jnp.zeros((SP,), jnp.int32)
        gg_w, bs_w = _run_gather(g, nbrs[wi], fidx_pad, batch, qw, last)
        if last:
            bsub = bs_w
        ps_pad = jnp.pad(
            jnp.stack([oxf, oyf, ozf], axis=1),
            ((0, sp_w - wlen), (0, 5)))
        outs.append(_run_mlp(gg_w, ps_pad, W1, b1, W2, b2, sp_w)[:wlen])
    pos_sub = jnp.stack([jnp.concatenate(xs), jnp.concatenate(ys),
                         jnp.concatenate(zs)], axis=1)
    out = jnp.concatenate(outs)
    return (out, pos_sub, bsub[:S])
